# baseline (speedup 1.0000x reference)
import jax
import jax.numpy as jnp
from jax import lax
from jax.experimental import pallas as pl
from jax.experimental.pallas import tpu as pltpu
from jax.experimental.pallas import tpu_sc as plsc

B, N, S, D = 8, 512, 256, 64
NB = 128

NC, NS, L = 2, 16, 16
NW = NC * NS
ROWS_PER_W = D // NW


def _sc_lookup_body(time_hbm, space_hbm, tt_hbm, st_hbm,
                    tchunk, schunk, rowbuf_t, rowbuf_s):
    wid = lax.axis_index("s") * NC + lax.axis_index("c")
    pltpu.sync_copy(time_hbm.at[pl.ds(0, S)], tchunk)
    pltpu.sync_copy(space_hbm, schunk)
    lane = lax.iota(jnp.int32, 16)
    for r in range(ROWS_PER_W):
        d = wid * ROWS_PER_W + r
        dvec = jnp.full((L,), d, jnp.int32)
        for i in range(S // L):
            rows = lane + i * L
            rowbuf_t[pl.ds(i * L, L)] = plsc.load_gather(tchunk, [rows, dvec])
        pltpu.sync_copy(rowbuf_t, tt_hbm.at[d])
        for i in range(N // L):
            rows = lane + i * L
            rowbuf_s[pl.ds(i * L, L)] = plsc.load_gather(schunk, [rows, dvec])
        pltpu.sync_copy(rowbuf_s, st_hbm.at[d])


_sc_lookup = pl.kernel(
    _sc_lookup_body,
    out_type=(
        jax.ShapeDtypeStruct((D, S), jnp.float32),
        jax.ShapeDtypeStruct((D, N), jnp.float32),
    ),
    mesh=plsc.VectorSubcoreMesh(
        core_axis_name="c", subcore_axis_name="s",
        num_cores=NC, num_subcores=NS,
    ),
    scratch_types=[
        pltpu.VMEM((S, D), jnp.float32),
        pltpu.VMEM((N, D), jnp.float32),
        pltpu.VMEM((S,), jnp.float32),
        pltpu.VMEM((N,), jnp.float32),
    ],
    compiler_params=pltpu.CompilerParams(needs_layout_passes=False),
)


def _tc_body(tt_ref, st_ref, out_ref):
    tt = tt_ref[...]
    st = st_ref[...]
    out_ref[0] = st[:, :, None] + tt[:, None, :]


def kernel(input_ids, time_table, space_table):
    del input_ids
    tt, st = _sc_lookup(time_table, space_table)
    grid = (B, N // NB)
    return pl.pallas_call(
        _tc_body,
        grid=grid,
        in_specs=[
            pl.BlockSpec((D, S), lambda b, j: (0, 0)),
            pl.BlockSpec((D, NB), lambda b, j: (0, j)),
        ],
        out_specs=pl.BlockSpec((1, D, NB, S), lambda b, j: (b, 0, j, 0)),
        out_shape=jax.ShapeDtypeStruct((B, D, N, S), jnp.float32),
    )(tt, st)
